# baseline (speedup 1.0000x reference)
import functools
import numpy as np
import jax
import jax.numpy as jnp
from jax import lax
from jax.experimental import pallas as pl
from jax.experimental.pallas import tpu as pltpu
from jax.experimental.pallas import tpu_sc as plsc

B = 256
C = 256
HW = 256
NUM_FIXED = 51
P_MEM = 0.1
NW = 32
SPW = B // NW
CHUNK = 64


def _threefry2x32(k0, k1, x0, x1):
    ks2 = k0 ^ k1 ^ jnp.uint32(0x1BD11BDA)
    ks = [k0, k1, ks2]
    rotations = [(13, 15, 26, 6), (17, 29, 16, 24)]
    x0 = x0 + k0
    x1 = x1 + k1
    for i in range(5):
        for r in rotations[i % 2]:
            x0 = x0 + x1
            x1 = (x1 << jnp.uint32(r)) | (x1 >> jnp.uint32(32 - r))
            x1 = x1 ^ x0
        x0 = x0 + ks[(i + 1) % 3]
        x1 = x1 + ks[(i + 2) % 3] + jnp.uint32(i + 1)
    return x0, x1


_GD = jax.lax.GatherDimensionNumbers(
    offset_dims=(), collapsed_slice_dims=(0,), start_index_map=(0,))


def _splat_lane(vec16, lane):
    idxs = jnp.full((16, 1), lane, jnp.int32)
    return jax.lax.gather(vec16, idxs, _GD, (1,),
                          mode=jax.lax.GatherScatterMode.PROMISE_IN_BOUNDS)


def _mask_vecs(idx_splat_u32):
    lane_iota = jax.lax.iota(jnp.uint32, 16)
    zero = lane_iota ^ lane_iota
    k0b, k1b = _threefry2x32(jnp.uint32(0), jnp.uint32(42), zero, idx_splat_u32)
    out = []
    for g in range(16):
        cvals = jnp.uint32(16 * g) + lane_iota
        j = cvals - jnp.uint32(NUM_FIXED)
        o0, o1 = _threefry2x32(k0b, k1b, zero, j)
        bits = o0 ^ o1
        u = jax.lax.bitcast_convert_type(
            (bits >> jnp.uint32(9)) | jnp.uint32(0x3F800000), jnp.float32) - 1.0
        kept = (cvals < jnp.uint32(NUM_FIXED)) | (u < P_MEM)
        out.append(jnp.where(kept, jnp.float32(1.0), jnp.float32(0.0)))
    return out


def _make_sc_call():
    mesh = plsc.VectorSubcoreMesh(core_axis_name="c", subcore_axis_name="s")

    @functools.partial(
        pl.kernel,
        mesh=mesh,
        out_type=jax.ShapeDtypeStruct((B, HW, C), jnp.float32),
        scratch_types=[
            pltpu.VMEM((B,), jnp.int32),
            pltpu.VMEM((CHUNK, C), jnp.float32),
        ],
    )
    def sc_kernel(x_hbm, idx_hbm, out_hbm, idx_v, buf):
        wid = lax.axis_index("s") * 2 + lax.axis_index("c")
        pltpu.sync_copy(idx_hbm, idx_v)
        for t in range(SPW):
            b = wid * SPW + t
            blk = (b // 16) * 16
            lane = b - blk
            vec16 = idx_v[pl.ds(blk, 16)]
            splat = _splat_lane(vec16, lane)
            masks = _mask_vecs(splat.astype(jnp.uint32))
            for h0 in range(0, HW, CHUNK):
                pltpu.sync_copy(x_hbm.at[b, pl.ds(h0, CHUNK)], buf)

                def row_body(r, carry):
                    for g in range(16):
                        v = buf[r, pl.ds(16 * g, 16)]
                        buf[r, pl.ds(16 * g, 16)] = v * masks[g]
                    return carry

                lax.fori_loop(0, CHUNK, row_body, 0)
                pltpu.sync_copy(buf, out_hbm.at[b, pl.ds(h0, CHUNK)])

    return sc_kernel


def kernel(X, idx):
    xt = jnp.transpose(X, (0, 2, 3, 1)).reshape(B, HW, C)
    out = _make_sc_call()(xt, idx)
    return jnp.transpose(out.reshape(B, 16, 16, C), (0, 3, 1, 2))


# speedup vs baseline: 2.6980x; 2.6980x over previous
import jax
import jax.numpy as jnp
from jax.experimental import pallas as pl
from jax.experimental.pallas import tpu as pltpu

B = 256
C = 256
HW = 256
NUM_FIXED = 51
P_MEM = 0.1
BBLK = 16


def _threefry2x32(k0, k1, x0, x1):
    ks2 = k0 ^ k1 ^ jnp.uint32(0x1BD11BDA)
    ks = [k0, k1, ks2]
    rotations = [(13, 15, 26, 6), (17, 29, 16, 24)]
    x0 = x0 + k0
    x1 = x1 + k1
    for i in range(5):
        for r in rotations[i % 2]:
            x0 = x0 + x1
            x1 = (x1 << jnp.uint32(r)) | (x1 >> jnp.uint32(32 - r))
            x1 = x1 ^ x0
        x0 = x0 + ks[(i + 1) % 3]
        x1 = x1 + ks[(i + 2) % 3] + jnp.uint32(i + 1)
    return x0, x1


def _mask_rows(idx_rows, nrows):
    i_u = jnp.broadcast_to(idx_rows.astype(jnp.uint32), (nrows, C))
    zero = jnp.zeros((nrows, C), jnp.uint32)
    k0b, k1b = _threefry2x32(jnp.uint32(0), jnp.uint32(42), zero, i_u)
    c = jax.lax.broadcasted_iota(jnp.uint32, (nrows, C), 1)
    j = c - jnp.uint32(NUM_FIXED)
    o0, o1 = _threefry2x32(k0b, k1b, zero, j)
    bits = o0 ^ o1
    u = jax.lax.bitcast_convert_type(
        (bits >> jnp.uint32(9)) | jnp.uint32(0x3F800000), jnp.float32) - 1.0
    kept = (c < jnp.uint32(NUM_FIXED)) | (u < P_MEM)
    return kept.astype(jnp.float32)


def _body(idx_ref, x_ref, o_ref):
    p = pl.program_id(0)
    rows = [
        jnp.full((1, C), idx_ref[p * BBLK + r], jnp.int32) for r in range(BBLK)
    ]
    idx_rows = jnp.concatenate(rows, axis=0)
    mask = _mask_rows(idx_rows, BBLK)
    o_ref[...] = x_ref[...] * mask[:, None, :]


def kernel(X, idx):
    xt = jnp.transpose(X, (0, 2, 3, 1)).reshape(B, HW, C)
    out = pl.pallas_call(
        _body,
        grid=(B // BBLK,),
        in_specs=[
            pl.BlockSpec(memory_space=pltpu.MemorySpace.SMEM),
            pl.BlockSpec((BBLK, HW, C), lambda i: (i, 0, 0)),
        ],
        out_specs=pl.BlockSpec((BBLK, HW, C), lambda i: (i, 0, 0)),
        out_shape=jax.ShapeDtypeStruct((B, HW, C), X.dtype),
    )(idx, xt)
    return jnp.transpose(out.reshape(B, 16, 16, C), (0, 3, 1, 2))


# speedup vs baseline: 2.8161x; 1.0437x over previous
import jax
import jax.numpy as jnp
from jax.experimental import pallas as pl
from jax.experimental.pallas import tpu as pltpu

B = 256
C = 256
HW = 256
NUM_FIXED = 51
P_MEM = 0.1
BBLK = 32


def _threefry2x32(k0, k1, x0, x1):
    ks2 = k0 ^ k1 ^ jnp.uint32(0x1BD11BDA)
    ks = [k0, k1, ks2]
    rotations = [(13, 15, 26, 6), (17, 29, 16, 24)]
    x0 = x0 + k0
    x1 = x1 + k1
    for i in range(5):
        for r in rotations[i % 2]:
            x0 = x0 + x1
            x1 = (x1 << jnp.uint32(r)) | (x1 >> jnp.uint32(32 - r))
            x1 = x1 ^ x0
        x0 = x0 + ks[(i + 1) % 3]
        x1 = x1 + ks[(i + 2) % 3] + jnp.uint32(i + 1)
    return x0, x1


def _mask_rows(idx_rows, nrows):
    i_u = jnp.broadcast_to(idx_rows.astype(jnp.uint32), (nrows, C))
    zero = jnp.zeros((nrows, C), jnp.uint32)
    k0b, k1b = _threefry2x32(jnp.uint32(0), jnp.uint32(42), zero, i_u)
    c = jax.lax.broadcasted_iota(jnp.uint32, (nrows, C), 1)
    j = c - jnp.uint32(NUM_FIXED)
    o0, o1 = _threefry2x32(k0b, k1b, zero, j)
    bits = o0 ^ o1
    u = jax.lax.bitcast_convert_type(
        (bits >> jnp.uint32(9)) | jnp.uint32(0x3F800000), jnp.float32) - 1.0
    kept = (c < jnp.uint32(NUM_FIXED)) | (u < P_MEM)
    return kept.astype(jnp.float32)


def _body(idx_ref, x_ref, o_ref):
    p = pl.program_id(0)
    rows = [
        jnp.full((1, C), idx_ref[p * BBLK + r], jnp.int32) for r in range(BBLK)
    ]
    idx_rows = jnp.concatenate(rows, axis=0)
    mask = _mask_rows(idx_rows, BBLK)
    o_ref[...] = x_ref[...] * mask[:, None, :]


def kernel(X, idx):
    xt = jnp.transpose(X, (0, 2, 3, 1)).reshape(B, HW, C)
    out = pl.pallas_call(
        _body,
        grid=(B // BBLK,),
        in_specs=[
            pl.BlockSpec(memory_space=pltpu.MemorySpace.SMEM),
            pl.BlockSpec((BBLK, HW, C), lambda i: (i, 0, 0)),
        ],
        out_specs=pl.BlockSpec((BBLK, HW, C), lambda i: (i, 0, 0)),
        out_shape=jax.ShapeDtypeStruct((B, HW, C), X.dtype),
        compiler_params=pltpu.CompilerParams(
            dimension_semantics=("parallel",)),
    )(idx, xt)
    return jnp.transpose(out.reshape(B, 16, 16, C), (0, 3, 1, 2))
